# Initial kernel scaffold; baseline (speedup 1.0000x reference)
import functools

import jax
import jax.numpy as jnp
from jax import lax
from jax.experimental import pallas as pl
from jax.experimental.pallas import tpu as pltpu
from jax.experimental.pallas import tpu_sc as plsc

B, L = 4096, 64
HIDDEN, OUT_DIM = 128, 256
LANES = 16
H_REGS = HIDDEN // LANES

NUM_CORES = 2
NUM_SUBCORES = 16
NW = NUM_CORES * NUM_SUBCORES
B_PER_W = B // NW

ROWS_PER_CHUNK = 2
IDS_PER_CHUNK = ROWS_PER_CHUNK * L
CHUNKS = B_PER_W // ROWS_PER_CHUNK
TOK_PER_W = B_PER_W * L


def _sc_pool_build():
    mesh = plsc.VectorSubcoreMesh(core_axis_name="c", subcore_axis_name="s")

    @functools.partial(
        pl.kernel,
        mesh=mesh,
        out_type=jax.ShapeDtypeStruct((B, HIDDEN), jnp.float32),
        scratch_types=[
            pltpu.VMEM((TOK_PER_W,), jnp.int32),
            pltpu.VMEM((TOK_PER_W,), jnp.float32),
            pltpu.VMEM((IDS_PER_CHUNK, HIDDEN), jnp.float32),
            pltpu.VMEM((IDS_PER_CHUNK, HIDDEN), jnp.float32),
            pltpu.VMEM((B_PER_W, HIDDEN), jnp.float32),
            pltpu.SemaphoreType.DMA,
            pltpu.SemaphoreType.DMA,
        ],
    )
    def sc_pool(ids_hbm, mask_hbm, table_hbm, out_hbm,
                idx_v, mask_v, rows0, rows1, pooled_v, sem0, sem1):
        wid = lax.axis_index("s") * NUM_CORES + lax.axis_index("c")
        tok_base = wid * TOK_PER_W

        pltpu.sync_copy(ids_hbm.at[pl.ds(tok_base, TOK_PER_W)], idx_v)
        pltpu.sync_copy(mask_hbm.at[pl.ds(tok_base, TOK_PER_W)], mask_v)

        def gather_start(chunk, buf, sem):
            cp = pltpu.make_async_copy(
                table_hbm.at[idx_v.at[pl.ds(chunk * IDS_PER_CHUNK, IDS_PER_CHUNK)]],
                buf, sem)
            cp.start()

        def gather_wait(buf, sem):
            pltpu.make_async_copy(
                table_hbm.at[idx_v.at[pl.ds(0, IDS_PER_CHUNK)]],
                buf, sem).wait()

        def pool_rows(chunk, buf):
            for r in range(ROWS_PER_CHUNK):
                row_local = chunk * ROWS_PER_CHUNK + r
                mask_off = row_local * L

                def tok_body(t4, carry):
                    accs = carry
                    for u in range(4):
                        t = t4 * 4 + u
                        m = plsc.load_gather(
                            mask_v, [jnp.full((LANES,), mask_off + t, jnp.int32)])
                        new = []
                        for h in range(H_REGS):
                            v = buf[r * L + t, pl.ds(h * LANES, LANES)]
                            new.append(accs[h] + v * m)
                        new.append(accs[H_REGS] + m)
                        accs = tuple(new)
                    return accs

                zero = jnp.zeros((LANES,), jnp.float32)
                init = tuple(zero for _ in range(H_REGS + 1))
                accs = lax.fori_loop(0, L // 4, tok_body, init)
                rinv = 1.0 / accs[H_REGS]
                for h in range(H_REGS):
                    pooled_v[row_local, pl.ds(h * LANES, LANES)] = accs[h] * rinv

        gather_start(0, rows0, sem0)

        def chunk_pair(i, _):
            gather_start(2 * i + 1, rows1, sem1)
            gather_wait(rows0, sem0)
            pool_rows(2 * i, rows0)
            nxt = jnp.minimum(2 * i + 2, CHUNKS - 1)
            gather_start(nxt, rows0, sem0)
            gather_wait(rows1, sem1)
            pool_rows(2 * i + 1, rows1)
            return 0

        lax.fori_loop(0, CHUNKS // 2, chunk_pair, 0)
        gather_wait(rows0, sem0)

        pltpu.sync_copy(pooled_v, out_hbm.at[pl.ds(wid * B_PER_W, B_PER_W)])

    return sc_pool


_sc_pool = _sc_pool_build()

_PROJ_BLOCK = 512


def _tc_proj_body(x_ref, w_ref, o_ref):
    x = x_ref[...]
    w = w_ref[...]
    y = lax.dot_general(x, w, (((1,), (1,)), ((), ())),
                        preferred_element_type=jnp.float32)
    ss = jnp.sum(y * y, axis=1, keepdims=True)
    norm = jnp.maximum(jnp.sqrt(ss), 1e-8)
    o_ref[...] = y / norm


def _tc_proj(pooled, proj_weight):
    return pl.pallas_call(
        _tc_proj_body,
        out_shape=jax.ShapeDtypeStruct((B, OUT_DIM), jnp.float32),
        grid=(B // _PROJ_BLOCK,),
        in_specs=[
            pl.BlockSpec((_PROJ_BLOCK, HIDDEN), lambda i: (i, 0)),
            pl.BlockSpec((OUT_DIM, HIDDEN), lambda i: (0, 0)),
        ],
        out_specs=pl.BlockSpec((_PROJ_BLOCK, OUT_DIM), lambda i: (i, 0)),
    )(pooled, proj_weight)


def kernel(input_ids, attention_mask, embedding_table, proj_weight):
    ids_flat = input_ids.reshape(-1)
    mask_flat = attention_mask.reshape(-1)
    pooled = _sc_pool(ids_flat, mask_flat, embedding_table)
    return _tc_proj(pooled, proj_weight)


# speedup vs baseline: 9.0122x; 9.0122x over previous
import functools

import jax
import jax.numpy as jnp
from jax import lax
from jax.experimental import pallas as pl
from jax.experimental.pallas import tpu as pltpu
from jax.experimental.pallas import tpu_sc as plsc

B, L = 4096, 64
HIDDEN, OUT_DIM = 128, 256
LANES = 16
H_REGS = HIDDEN // LANES

NUM_CORES = 2
NUM_SUBCORES = 16
NW = NUM_CORES * NUM_SUBCORES
B_PER_W = B // NW

ROWS_PER_CHUNK = 2
IDS_PER_CHUNK = ROWS_PER_CHUNK * L
CHUNKS = B_PER_W // ROWS_PER_CHUNK
TOK_PER_W = B_PER_W * L


def _sc_pool_build():
    mesh = plsc.VectorSubcoreMesh(core_axis_name="c", subcore_axis_name="s")

    @functools.partial(
        pl.kernel,
        mesh=mesh,
        out_type=jax.ShapeDtypeStruct((B, HIDDEN), jnp.float32),
        scratch_types=[
            pltpu.VMEM((TOK_PER_W,), jnp.int32),
            pltpu.VMEM((TOK_PER_W,), jnp.float32),
            pltpu.VMEM((IDS_PER_CHUNK, HIDDEN), jnp.float32),
            pltpu.VMEM((IDS_PER_CHUNK, HIDDEN), jnp.float32),
            pltpu.VMEM((B_PER_W, HIDDEN), jnp.float32),
            pltpu.SemaphoreType.DMA,
            pltpu.SemaphoreType.DMA,
        ],
    )
    def sc_pool(ids_hbm, mask_hbm, table_hbm, out_hbm,
                idx_v, mask_v, rows0, rows1, pooled_v, sem0, sem1):
        wid = lax.axis_index("s") * NUM_CORES + lax.axis_index("c")
        tok_base = wid * TOK_PER_W

        pltpu.sync_copy(ids_hbm.at[pl.ds(tok_base, TOK_PER_W)], idx_v)
        pltpu.sync_copy(mask_hbm.at[pl.ds(tok_base, TOK_PER_W)], mask_v)

        def gather_start(chunk, buf, sem):
            cp = pltpu.make_async_copy(
                table_hbm.at[idx_v.at[pl.ds(chunk * IDS_PER_CHUNK, IDS_PER_CHUNK)]],
                buf, sem)
            cp.start()

        def gather_wait(buf, sem):
            pltpu.make_async_copy(
                table_hbm.at[idx_v.at[pl.ds(0, IDS_PER_CHUNK)]],
                buf, sem).wait()

        def pool_rows(chunk, buf):
            for r in range(ROWS_PER_CHUNK):
                row_local = chunk * ROWS_PER_CHUNK + r
                mask_off = row_local * L

                def tok_body(g, carry):
                    accs = carry
                    m16 = mask_v[pl.ds(mask_off + g * LANES, LANES)]
                    new = list(accs)
                    for u in range(LANES):
                        t = g * LANES + u
                        m = m16[u]
                        for h in range(H_REGS):
                            v = buf[r * L + t, pl.ds(h * LANES, LANES)]
                            new[h] = new[h] + v * m
                        new[H_REGS] = new[H_REGS] + m
                    return tuple(new)

                zero = jnp.zeros((LANES,), jnp.float32)
                init = tuple(zero for _ in range(H_REGS)) + (jnp.float32(0.0),)
                accs = lax.fori_loop(0, L // LANES, tok_body, init)
                rinv = 1.0 / jnp.full((LANES,), accs[H_REGS], jnp.float32)
                for h in range(H_REGS):
                    pooled_v[row_local, pl.ds(h * LANES, LANES)] = accs[h] * rinv

        gather_start(0, rows0, sem0)

        def chunk_pair(i, _):
            gather_start(2 * i + 1, rows1, sem1)
            gather_wait(rows0, sem0)
            pool_rows(2 * i, rows0)
            nxt = jnp.minimum(2 * i + 2, CHUNKS - 1)
            gather_start(nxt, rows0, sem0)
            gather_wait(rows1, sem1)
            pool_rows(2 * i + 1, rows1)
            return 0

        lax.fori_loop(0, CHUNKS // 2, chunk_pair, 0)
        gather_wait(rows0, sem0)

        pltpu.sync_copy(pooled_v, out_hbm.at[pl.ds(wid * B_PER_W, B_PER_W)])

    return sc_pool


_sc_pool = _sc_pool_build()

_PROJ_BLOCK = 512


def _tc_proj_body(x_ref, w_ref, o_ref):
    x = x_ref[...]
    w = w_ref[...]
    y = lax.dot_general(x, w, (((1,), (1,)), ((), ())),
                        preferred_element_type=jnp.float32)
    ss = jnp.sum(y * y, axis=1, keepdims=True)
    norm = jnp.maximum(jnp.sqrt(ss), 1e-8)
    o_ref[...] = y / norm


def _tc_proj(pooled, proj_weight):
    return pl.pallas_call(
        _tc_proj_body,
        out_shape=jax.ShapeDtypeStruct((B, OUT_DIM), jnp.float32),
        grid=(B // _PROJ_BLOCK,),
        in_specs=[
            pl.BlockSpec((_PROJ_BLOCK, HIDDEN), lambda i: (i, 0)),
            pl.BlockSpec((OUT_DIM, HIDDEN), lambda i: (0, 0)),
        ],
        out_specs=pl.BlockSpec((_PROJ_BLOCK, OUT_DIM), lambda i: (i, 0)),
    )(pooled, proj_weight)


def kernel(input_ids, attention_mask, embedding_table, proj_weight):
    ids_flat = input_ids.reshape(-1)
    mask_flat = attention_mask.reshape(-1)
    pooled = _sc_pool(ids_flat, mask_flat, embedding_table)
    return _tc_proj(pooled, proj_weight)
